# baseline (speedup 1.0000x reference)
import functools

import jax
import jax.numpy as jnp
from jax import lax
from jax.experimental import pallas as pl
from jax.experimental.pallas import tpu as pltpu
from jax.experimental.pallas import tpu_sc as plsc

N = 10000
E = 320000
H = 128

NC = 2
NS = 16
NW = NC * NS
CHUNK = 128
CH_PER_W = 80
CHUNKS_TOTAL = CH_PER_W * NW
E_PAD = CHUNKS_TOTAL * CHUNK
AGG_ROWS = N + NS
ZROWS = 632
OROWS = 624
Q = 16
NQ = CH_PER_W // Q


def _sc_segsum_body(ff_hbm, src_hbm, dst_hbm, zeros_hbm, out_hbm,
                    src_q0, src_q1, dst_q0, dst_q1, rows_a, rows_b, agg_s,
                    sem_a, sem_b, sem_i, sem_z):
    c = lax.axis_index("c")
    s = lax.axis_index("s")
    w = c * NS + s
    @pl.when(s < NS - 1)
    def _():
        pltpu.async_copy(zeros_hbm, agg_s.at[pl.ds(s * ZROWS, ZROWS)], sem_z)

    @pl.when(s == NS - 1)
    def _():
        tail = AGG_ROWS - (NS - 1) * ZROWS
        pltpu.async_copy(zeros_hbm.at[pl.ds(0, tail)],
                         agg_s.at[pl.ds((NS - 1) * ZROWS, tail)], sem_z)

    base = w * CH_PER_W
    sbufs = (src_q0, src_q1)
    dbufs = (dst_q0, dst_q1)

    def idx_start(q, sb, db):
        off = base + q * Q
        pltpu.async_copy(src_hbm.at[pl.ds(off, Q)], sb, sem_i)
        pltpu.async_copy(dst_hbm.at[pl.ds(off, Q)], db, sem_i)

    def idx_wait(q, sb, db):
        off = base + q * Q
        pltpu.make_async_copy(src_hbm.at[pl.ds(off, Q)], sb, sem_i).wait()
        pltpu.make_async_copy(dst_hbm.at[pl.ds(off, Q)], db, sem_i).wait()

    pltpu.sync_copy(src_hbm.at[pl.ds(base, Q)], src_q0)
    pltpu.sync_copy(dst_hbm.at[pl.ds(base, Q)], dst_q0)
    idx_start(1, src_q1, dst_q1)
    pltpu.async_copy(ff_hbm.at[src_q0.at[0]], rows_a, sem_a)
    pltpu.async_copy(ff_hbm.at[src_q0.at[1]], rows_b, sem_b)

    @pl.when(s < NS - 1)
    def _():
        pltpu.make_async_copy(
            zeros_hbm, agg_s.at[pl.ds(s * ZROWS, ZROWS)], sem_z).wait()

    @pl.when(s == NS - 1)
    def _():
        tail = AGG_ROWS - (NS - 1) * ZROWS
        pltpu.make_async_copy(
            zeros_hbm.at[pl.ds(0, tail)],
            agg_s.at[pl.ds((NS - 1) * ZROWS, tail)], sem_z).wait()

    plsc.subcore_barrier()

    for q in range(NQ):
        cur_s, cur_d = sbufs[q % 2], dbufs[q % 2]
        nxt_s, nxt_d = sbufs[(q + 1) % 2], dbufs[(q + 1) % 2]

        def step(i, carry, cur_s=cur_s, cur_d=cur_d):
            j = 2 * i
            pltpu.make_async_copy(ff_hbm.at[cur_s.at[j]], rows_a, sem_a).wait()
            pltpu.sync_copy(rows_a, agg_s.at[cur_d.at[j]], add=True)
            pltpu.async_copy(ff_hbm.at[cur_s.at[j + 2]], rows_a, sem_a)
            pltpu.make_async_copy(ff_hbm.at[cur_s.at[j + 1]], rows_b, sem_b).wait()
            pltpu.sync_copy(rows_b, agg_s.at[cur_d.at[j + 1]], add=True)
            pltpu.async_copy(ff_hbm.at[cur_s.at[j + 3]], rows_b, sem_b)
            return carry

        lax.fori_loop(0, Q // 2 - 1, step, 0)
        j = Q - 2
        if q < NQ - 1:
            idx_wait(q + 1, nxt_s, nxt_d)
            pltpu.make_async_copy(ff_hbm.at[cur_s.at[j]], rows_a, sem_a).wait()
            pltpu.sync_copy(rows_a, agg_s.at[cur_d.at[j]], add=True)
            pltpu.async_copy(ff_hbm.at[nxt_s.at[0]], rows_a, sem_a)
            pltpu.make_async_copy(ff_hbm.at[cur_s.at[j + 1]], rows_b, sem_b).wait()
            pltpu.sync_copy(rows_b, agg_s.at[cur_d.at[j + 1]], add=True)
            pltpu.async_copy(ff_hbm.at[nxt_s.at[1]], rows_b, sem_b)
            if q + 2 < NQ:
                idx_start(q + 2, cur_s, cur_d)
        else:
            pltpu.make_async_copy(ff_hbm.at[cur_s.at[j]], rows_a, sem_a).wait()
            pltpu.sync_copy(rows_a, agg_s.at[cur_d.at[j]], add=True)
            pltpu.make_async_copy(ff_hbm.at[cur_s.at[j + 1]], rows_b, sem_b).wait()
            pltpu.sync_copy(rows_b, agg_s.at[cur_d.at[j + 1]], add=True)
    plsc.subcore_barrier()
    pltpu.sync_copy(agg_s.at[pl.ds(s * OROWS, OROWS)],
                    out_hbm.at[c, pl.ds(s * OROWS, OROWS)])

    @pl.when(s == 0)
    def _():
        pltpu.sync_copy(agg_s.at[pl.ds(NS * OROWS, N - NS * OROWS)],
                        out_hbm.at[c, pl.ds(NS * OROWS, N - NS * OROWS)])


_sc_segsum = pl.kernel(
    _sc_segsum_body,
    out_type=jax.ShapeDtypeStruct((NC, N, H), jnp.float32),
    mesh=plsc.VectorSubcoreMesh(core_axis_name="c", subcore_axis_name="s"),
    scratch_types=[
        pltpu.VMEM((Q, CHUNK), jnp.int32),
        pltpu.VMEM((Q, CHUNK), jnp.int32),
        pltpu.VMEM((Q, CHUNK), jnp.int32),
        pltpu.VMEM((Q, CHUNK), jnp.int32),
        pltpu.VMEM((CHUNK, H), jnp.float32),
        pltpu.VMEM((CHUNK, H), jnp.float32),
        pltpu.VMEM_SHARED((AGG_ROWS, H), jnp.float32),
        pltpu.SemaphoreType.DMA,
        pltpu.SemaphoreType.DMA,
        pltpu.SemaphoreType.DMA,
        pltpu.SemaphoreType.DMA,
    ],
)


def _tc_ff_body(x_ref, w_ref, b_ref, g_ref, bb_ref, o_ref):
    x = x_ref[...]
    h = jnp.maximum(_mm_t(x, w_ref[...]) + b_ref[...], 0.0) + x
    o_ref[...] = _ln(h, g_ref[...], bb_ref[...])


def _ln(h, g, b):
    m = jnp.mean(h, axis=-1, keepdims=True)
    v = jnp.mean(jnp.square(h - m), axis=-1, keepdims=True)
    return (h - m) * lax.rsqrt(v + 1e-5) * g + b


def _mm_t(x, w):
    return lax.dot_general(x, w, (((1,), (1,)), ((), ())),
                           preferred_element_type=jnp.float32)


def _tc_out_body(p_ref, ff_ref, w_ref, b_ref, g_ref, bb_ref, o_ref):
    a = p_ref[0] + p_ref[1]
    h = jnp.maximum(_mm_t(a, w_ref[...]) + b_ref[...], 0.0) + ff_ref[...]
    o_ref[...] = _ln(h, g_ref[...], bb_ref[...])


def _tc_out_ff_body(p_ref, ff_ref, gw_ref, gb_ref, lg_ref, lb_ref,
                    w_ref, b_ref, fg_ref, fb_ref, o_ref):
    a = p_ref[0] + p_ref[1]
    h = jnp.maximum(_mm_t(a, gw_ref[...]) + gb_ref[...], 0.0) + ff_ref[...]
    out = _ln(h, lg_ref[...], lb_ref[...])
    h2 = jnp.maximum(_mm_t(out, w_ref[...]) + b_ref[...], 0.0) + out
    o_ref[...] = _ln(h2, fg_ref[...], fb_ref[...])


_ROWS_BLK = 10000
_GRID = N // _ROWS_BLK

_vec_spec = pl.BlockSpec((1, H), lambda i: (0, 0))
_mat_spec = pl.BlockSpec((H, H), lambda i: (0, 0))
_rows_spec = pl.BlockSpec((_ROWS_BLK, H), lambda i: (i, 0))

_tc_ff = pl.pallas_call(
    _tc_ff_body,
    grid=(_GRID,),
    in_specs=[_rows_spec, _mat_spec, _vec_spec, _vec_spec, _vec_spec],
    out_specs=_rows_spec,
    out_shape=jax.ShapeDtypeStruct((N, H), jnp.float32),
)

_tc_out = pl.pallas_call(
    _tc_out_body,
    grid=(_GRID,),
    in_specs=[pl.BlockSpec((NC, _ROWS_BLK, H), lambda i: (0, i, 0)),
              _rows_spec, _mat_spec, _vec_spec, _vec_spec, _vec_spec],
    out_specs=_rows_spec,
    out_shape=jax.ShapeDtypeStruct((N, H), jnp.float32),
)

_tc_out_ff = pl.pallas_call(
    _tc_out_ff_body,
    grid=(_GRID,),
    in_specs=[pl.BlockSpec((NC, _ROWS_BLK, H), lambda i: (0, i, 0)),
              _rows_spec, _mat_spec, _vec_spec, _vec_spec, _vec_spec,
              _mat_spec, _vec_spec, _vec_spec, _vec_spec],
    out_specs=_rows_spec,
    out_shape=jax.ShapeDtypeStruct((N, H), jnp.float32),
)


def kernel(x, W_lin, b_lin, gcn0_W, gcn0_b, gcn1_W, gcn1_b,
           ffln0_g, ffln0_b, ffln1_g, ffln1_b,
           ln0_g, ln0_b, ln1_g, ln1_b, edge_index):
    npad = E_PAD - E
    pad_src = (jnp.arange(npad, dtype=jnp.int32) % H)
    pad_dst = N + (jnp.arange(npad, dtype=jnp.int32) % NS)
    src2d = jnp.concatenate([edge_index[0], pad_src]).reshape(CHUNKS_TOTAL, CHUNK)
    dst2d = jnp.concatenate([edge_index[1], pad_dst]).reshape(CHUNKS_TOTAL, CHUNK)
    zeros = jnp.zeros((ZROWS, H), jnp.float32)

    r = lambda v: v.reshape(1, H)
    ff0 = _tc_ff(x, W_lin, r(b_lin), r(ffln0_g), r(ffln0_b))
    p0 = _sc_segsum(ff0, src2d, dst2d, zeros)
    ff1 = _tc_out_ff(p0, ff0, gcn0_W, r(gcn0_b), r(ln0_g), r(ln0_b),
                     W_lin, r(b_lin), r(ffln1_g), r(ffln1_b))
    p1 = _sc_segsum(ff1, src2d, dst2d, zeros)
    return _tc_out(p1, ff1, gcn1_W, r(gcn1_b), r(ln1_g), r(ln1_b))


# speedup vs baseline: 1.0181x; 1.0181x over previous
import jax
import jax.numpy as jnp
from jax import lax
from jax.experimental import pallas as pl
from jax.experimental.pallas import tpu as pltpu
from jax.experimental.pallas import tpu_sc as plsc

N = 10000
E = 320000
H = 128

NC = 2
NS = 16
NW = NC * NS
CHUNK = 128
CH_PER_W = 80
CHUNKS_TOTAL = CH_PER_W * NW
E_PAD = CHUNKS_TOTAL * CHUNK
AGG_ROWS = N + NS
ZROWS = 632
OROWS = 624
Q = 16
NQ = CH_PER_W // Q


def _sc_segsum_body(ff_hbm, src_hbm, dst_hbm, zeros_hbm, out_hbm,
                    src_q0, src_q1, dst_q0, dst_q1, rows_a, rows_b, agg_s,
                    sem_a, sem_b, sem_i, sem_z):
    c = lax.axis_index("c")
    s = lax.axis_index("s")
    w = c * NS + s
    @pl.when(s < NS - 1)
    def _():
        pltpu.async_copy(zeros_hbm, agg_s.at[pl.ds(s * ZROWS, ZROWS)], sem_z)

    @pl.when(s == NS - 1)
    def _():
        tail = AGG_ROWS - (NS - 1) * ZROWS
        pltpu.async_copy(zeros_hbm.at[pl.ds(0, tail)],
                         agg_s.at[pl.ds((NS - 1) * ZROWS, tail)], sem_z)

    base = w * CH_PER_W
    sbufs = (src_q0, src_q1)
    dbufs = (dst_q0, dst_q1)

    def idx_start(q, sb, db):
        off = base + q * Q
        pltpu.async_copy(src_hbm.at[pl.ds(off, Q)], sb, sem_i)
        pltpu.async_copy(dst_hbm.at[pl.ds(off, Q)], db, sem_i)

    def idx_wait(q, sb, db):
        off = base + q * Q
        pltpu.make_async_copy(src_hbm.at[pl.ds(off, Q)], sb, sem_i).wait()
        pltpu.make_async_copy(dst_hbm.at[pl.ds(off, Q)], db, sem_i).wait()

    pltpu.sync_copy(src_hbm.at[pl.ds(base, Q)], src_q0)
    pltpu.sync_copy(dst_hbm.at[pl.ds(base, Q)], dst_q0)
    idx_start(1, src_q1, dst_q1)
    pltpu.async_copy(ff_hbm.at[src_q0.at[0]], rows_a, sem_a)
    pltpu.async_copy(ff_hbm.at[src_q0.at[1]], rows_b, sem_b)

    @pl.when(s < NS - 1)
    def _():
        pltpu.make_async_copy(
            zeros_hbm, agg_s.at[pl.ds(s * ZROWS, ZROWS)], sem_z).wait()

    @pl.when(s == NS - 1)
    def _():
        tail = AGG_ROWS - (NS - 1) * ZROWS
        pltpu.make_async_copy(
            zeros_hbm.at[pl.ds(0, tail)],
            agg_s.at[pl.ds((NS - 1) * ZROWS, tail)], sem_z).wait()

    plsc.subcore_barrier()

    for q in range(NQ):
        cur_s, cur_d = sbufs[q % 2], dbufs[q % 2]
        nxt_s, nxt_d = sbufs[(q + 1) % 2], dbufs[(q + 1) % 2]

        def step(i, carry, cur_s=cur_s, cur_d=cur_d):
            j = 2 * i
            pltpu.make_async_copy(ff_hbm.at[cur_s.at[j]], rows_a, sem_a).wait()
            pltpu.sync_copy(rows_a, agg_s.at[cur_d.at[j]], add=True)
            pltpu.async_copy(ff_hbm.at[cur_s.at[j + 2]], rows_a, sem_a)
            pltpu.make_async_copy(ff_hbm.at[cur_s.at[j + 1]], rows_b, sem_b).wait()
            pltpu.sync_copy(rows_b, agg_s.at[cur_d.at[j + 1]], add=True)
            pltpu.async_copy(ff_hbm.at[cur_s.at[j + 3]], rows_b, sem_b)
            return carry

        lax.fori_loop(0, Q // 2 - 1, step, 0)
        j = Q - 2
        if q < NQ - 1:
            idx_wait(q + 1, nxt_s, nxt_d)
            pltpu.make_async_copy(ff_hbm.at[cur_s.at[j]], rows_a, sem_a).wait()
            pltpu.sync_copy(rows_a, agg_s.at[cur_d.at[j]], add=True)
            pltpu.async_copy(ff_hbm.at[nxt_s.at[0]], rows_a, sem_a)
            pltpu.make_async_copy(ff_hbm.at[cur_s.at[j + 1]], rows_b, sem_b).wait()
            pltpu.sync_copy(rows_b, agg_s.at[cur_d.at[j + 1]], add=True)
            pltpu.async_copy(ff_hbm.at[nxt_s.at[1]], rows_b, sem_b)
            if q + 2 < NQ:
                idx_start(q + 2, cur_s, cur_d)
        else:
            pltpu.make_async_copy(ff_hbm.at[cur_s.at[j]], rows_a, sem_a).wait()
            pltpu.sync_copy(rows_a, agg_s.at[cur_d.at[j]], add=True)
            pltpu.make_async_copy(ff_hbm.at[cur_s.at[j + 1]], rows_b, sem_b).wait()
            pltpu.sync_copy(rows_b, agg_s.at[cur_d.at[j + 1]], add=True)
    plsc.subcore_barrier()
    pltpu.sync_copy(agg_s.at[pl.ds(s * OROWS, OROWS)],
                    out_hbm.at[c, pl.ds(s * OROWS, OROWS)])

    @pl.when(s == 0)
    def _():
        pltpu.sync_copy(agg_s.at[pl.ds(NS * OROWS, N - NS * OROWS)],
                        out_hbm.at[c, pl.ds(NS * OROWS, N - NS * OROWS)])


_sc_segsum = pl.kernel(
    _sc_segsum_body,
    out_type=jax.ShapeDtypeStruct((NC, N, H), jnp.float32),
    mesh=plsc.VectorSubcoreMesh(core_axis_name="c", subcore_axis_name="s"),
    scratch_types=[
        pltpu.VMEM((Q, CHUNK), jnp.int32),
        pltpu.VMEM((Q, CHUNK), jnp.int32),
        pltpu.VMEM((Q, CHUNK), jnp.int32),
        pltpu.VMEM((Q, CHUNK), jnp.int32),
        pltpu.VMEM((CHUNK, H), jnp.float32),
        pltpu.VMEM((CHUNK, H), jnp.float32),
        pltpu.VMEM_SHARED((AGG_ROWS, H), jnp.float32),
        pltpu.SemaphoreType.DMA,
        pltpu.SemaphoreType.DMA,
        pltpu.SemaphoreType.DMA,
        pltpu.SemaphoreType.DMA,
    ],
)


def _tc_ff_body(x_ref, w_ref, b_ref, g_ref, bb_ref, o_ref):
    x = x_ref[...]
    h = jnp.maximum(_mm_t(x, w_ref[...]) + b_ref[...], 0.0) + x
    o_ref[...] = _ln(h, g_ref[...], bb_ref[...])


def _ln(h, g, b):
    m = jnp.mean(h, axis=-1, keepdims=True)
    v = jnp.mean(jnp.square(h - m), axis=-1, keepdims=True)
    return (h - m) * lax.rsqrt(v + 1e-5) * g + b


def _mm_t(x, w):
    return lax.dot_general(x, w, (((1,), (1,)), ((), ())),
                           preferred_element_type=jnp.float32)


def _tc_out_body(p_ref, ff_ref, w_ref, b_ref, g_ref, bb_ref, o_ref):
    a = p_ref[0] + p_ref[1]
    h = jnp.maximum(_mm_t(a, w_ref[...]) + b_ref[...], 0.0) + ff_ref[...]
    o_ref[...] = _ln(h, g_ref[...], bb_ref[...])


def _tc_out_ff_body(p_ref, ff_ref, gw_ref, gb_ref, lg_ref, lb_ref,
                    w_ref, b_ref, fg_ref, fb_ref, o_ref):
    a = p_ref[0] + p_ref[1]
    h = jnp.maximum(_mm_t(a, gw_ref[...]) + gb_ref[...], 0.0) + ff_ref[...]
    out = _ln(h, lg_ref[...], lb_ref[...])
    h2 = jnp.maximum(_mm_t(out, w_ref[...]) + b_ref[...], 0.0) + out
    o_ref[...] = _ln(h2, fg_ref[...], fb_ref[...])


_ROWS_BLK = 5000
_GRID = N // _ROWS_BLK

_vec_spec = pl.BlockSpec((1, H), lambda i: (0, 0))
_mat_spec = pl.BlockSpec((H, H), lambda i: (0, 0))
_rows_spec = pl.BlockSpec((_ROWS_BLK, H), lambda i: (i, 0))

_tc_ff = pl.pallas_call(
    _tc_ff_body,
    grid=(_GRID,),
    in_specs=[_rows_spec, _mat_spec, _vec_spec, _vec_spec, _vec_spec],
    out_specs=_rows_spec,
    out_shape=jax.ShapeDtypeStruct((N, H), jnp.float32),
)

_tc_out = pl.pallas_call(
    _tc_out_body,
    grid=(_GRID,),
    in_specs=[pl.BlockSpec((NC, _ROWS_BLK, H), lambda i: (0, i, 0)),
              _rows_spec, _mat_spec, _vec_spec, _vec_spec, _vec_spec],
    out_specs=_rows_spec,
    out_shape=jax.ShapeDtypeStruct((N, H), jnp.float32),
)

_tc_out_ff = pl.pallas_call(
    _tc_out_ff_body,
    grid=(_GRID,),
    in_specs=[pl.BlockSpec((NC, _ROWS_BLK, H), lambda i: (0, i, 0)),
              _rows_spec, _mat_spec, _vec_spec, _vec_spec, _vec_spec,
              _mat_spec, _vec_spec, _vec_spec, _vec_spec],
    out_specs=_rows_spec,
    out_shape=jax.ShapeDtypeStruct((N, H), jnp.float32),
)


def kernel(x, W_lin, b_lin, gcn0_W, gcn0_b, gcn1_W, gcn1_b,
           ffln0_g, ffln0_b, ffln1_g, ffln1_b,
           ln0_g, ln0_b, ln1_g, ln1_b, edge_index):
    npad = E_PAD - E
    pad_src = (jnp.arange(npad, dtype=jnp.int32) % H)
    pad_dst = N + (jnp.arange(npad, dtype=jnp.int32) % NS)
    src2d = jnp.concatenate([edge_index[0], pad_src]).reshape(CHUNKS_TOTAL, CHUNK)
    dst2d = jnp.concatenate([edge_index[1], pad_dst]).reshape(CHUNKS_TOTAL, CHUNK)
    zeros = jnp.zeros((ZROWS, H), jnp.float32)

    r = lambda v: v.reshape(1, H)
    ff0 = _tc_ff(x, W_lin, r(b_lin), r(ffln0_g), r(ffln0_b))
    p0 = _sc_segsum(ff0, src2d, dst2d, zeros)
    ff1 = _tc_out_ff(p0, ff0, gcn0_W, r(gcn0_b), r(ln0_g), r(ln0_b),
                     W_lin, r(b_lin), r(ffln1_g), r(ffln1_b))
    p1 = _sc_segsum(ff1, src2d, dst2d, zeros)
    return _tc_out(p1, ff1, gcn1_W, r(gcn1_b), r(ln1_g), r(ln1_b))
